# Initial kernel scaffold; baseline (speedup 1.0000x reference)
import jax
import jax.numpy as jnp
from jax.experimental import pallas as pl


def kernel(rpn_probs, rpn_bbox, anchors):
    raise NotImplementedError("write your pallas kernel here")


# speedup vs baseline: 40.5479x; 40.5479x over previous
import functools

import jax
import jax.numpy as jnp
from jax import lax
from jax.experimental import pallas as pl
from jax.experimental.pallas import tpu as pltpu
from jax.experimental.pallas import tpu_sc as plsc

_B = 2
_N = 20000
_PROPOSALS = 2000
_TH = 0.7
_PRE = 6000
_PADN = 6144
_T = 256
_NB = _PADN // _T
_STD = (0.1, 0.1, 0.2, 0.2)


def _sc_gather(table, idx):
    info = plsc.get_sparse_core_info()
    nw = info.num_cores * info.num_subcores
    btot = idx.shape[0]
    bpw = btot // nw
    lanes = table.shape[1]
    mesh = plsc.VectorSubcoreMesh(core_axis_name="c", subcore_axis_name="s")

    @functools.partial(
        pl.kernel,
        mesh=mesh,
        out_type=jax.ShapeDtypeStruct((btot, lanes), jnp.float32),
        scratch_types=[
            pltpu.VMEM((bpw,), jnp.int32),
            pltpu.VMEM((bpw, lanes), jnp.float32),
            pltpu.SemaphoreType.DMA,
        ],
    )
    def k(table_hbm, idx_hbm, out_hbm, idx_v, rows_v, sem):
        wid = lax.axis_index("s") * info.num_cores + lax.axis_index("c")
        base = wid * bpw
        pltpu.sync_copy(idx_hbm.at[pl.ds(base, bpw)], idx_v)
        pltpu.async_copy(table_hbm.at[idx_v], rows_v, sem).wait()
        pltpu.sync_copy(rows_v, out_hbm.at[pl.ds(base, bpw)])

    return k(table, idx)


def _nms_kernel(g_ref, out_ref, y1s, x1s, y2s, x2s, ars, kps, iou_s,
                oy1, ox1, oy2, ox2):
    g = g_ref[0]
    y1 = jnp.clip(g[:, 0:1], 0.0, 1.0)
    x1 = jnp.clip(g[:, 1:2], 0.0, 1.0)
    y2 = jnp.clip(g[:, 2:3], 0.0, 1.0)
    x2 = jnp.clip(g[:, 3:4], 0.0, 1.0)
    area = (y2 - y1) * (x2 - x1)
    y1s[...] = y1
    x1s[...] = x1
    y2s[...] = y2
    x2s[...] = x2
    ars[...] = area

    sub = lax.broadcasted_iota(jnp.int32, (_PADN, 1), 0)
    kps[...] = jnp.where(sub < _PRE, 1.0, 0.0)
    zcol = jnp.zeros((_PROPOSALS, 1), jnp.float32)
    oy1[...] = zcol
    ox1[...] = zcol
    oy2[...] = zcol
    ox2[...] = zcol

    lane = lax.broadcasted_iota(jnp.int32, (1, _T), 1)
    r2000 = lax.broadcasted_iota(
        jnp.int32, (_PROPOSALS, 1), 0).astype(jnp.float32)
    csm = jnp.where(
        lax.broadcasted_iota(jnp.int32, (_T, _T), 0)
        <= lax.broadcasted_iota(jnp.int32, (_T, _T), 1), 1.0, 0.0)
    dn = (((1,), (0,)), ((), ()))

    for b in range(_NB):
        r0 = b * _T
        cnt = jnp.sum(jnp.where(sub < r0, kps[...], 0.0))

        @pl.when(cnt < float(_PROPOSALS))
        def _block(b=b, r0=r0, cnt=cnt):
            by1 = y1s[r0:r0 + _T, :]
            bx1 = x1s[r0:r0 + _T, :]
            by2 = y2s[r0:r0 + _T, :]
            bx2 = x2s[r0:r0 + _T, :]
            bar = ars[r0:r0 + _T, :]
            ry1 = jnp.transpose(by1)
            rx1 = jnp.transpose(bx1)
            ry2 = jnp.transpose(by2)
            rx2 = jnp.transpose(bx2)
            rar = jnp.transpose(bar)

            def chunk_body(c, sup):
                c0 = c * _T
                cy1 = y1s[pl.ds(c0, _T), :]
                cx1 = x1s[pl.ds(c0, _T), :]
                cy2 = y2s[pl.ds(c0, _T), :]
                cx2 = x2s[pl.ds(c0, _T), :]
                car = ars[pl.ds(c0, _T), :]
                ckp = kps[pl.ds(c0, _T), :]
                yy1 = jnp.maximum(cy1, ry1)
                xx1 = jnp.maximum(cx1, rx1)
                yy2 = jnp.minimum(cy2, ry2)
                xx2 = jnp.minimum(cx2, rx2)
                inter = (jnp.maximum(yy2 - yy1, 0.0)
                         * jnp.maximum(xx2 - xx1, 0.0))
                union = car + rar - inter
                iou = inter / (union + 1e-9)
                hit = jnp.where((iou > _TH) & (ckp > 0.0), 1.0, 0.0)
                return jnp.maximum(sup, jnp.max(hit, axis=0, keepdims=True))

            sup = jnp.zeros((1, _T), jnp.float32)
            if b > 0:
                sup = lax.fori_loop(0, b, chunk_body, sup)

            valid = jnp.transpose(kps[r0:r0 + _T, :])
            keep0 = valid * (1.0 - sup)

            yy1 = jnp.maximum(by1, ry1)
            xx1 = jnp.maximum(bx1, rx1)
            yy2 = jnp.minimum(by2, ry2)
            xx2 = jnp.minimum(bx2, rx2)
            inter = jnp.maximum(yy2 - yy1, 0.0) * jnp.maximum(xx2 - xx1, 0.0)
            union = bar + rar - inter
            iou_s[...] = inter / (union + 1e-9)

            def intra(i, kr):
                row = iou_s[pl.ds(i, 1), :]
                ki = jnp.sum(jnp.where(lane == i, kr, 0.0))
                su = jnp.where((row > _TH) & (lane > i), ki, 0.0)
                return kr * (1.0 - su)

            kr = lax.fori_loop(0, _T, intra, keep0)
            kps[r0:r0 + _T, :] = jnp.transpose(kr)

            csum = lax.dot_general(kr, csm, dn,
                                   preferred_element_type=jnp.float32)
            ranks = (cnt + csum) - kr
            p = jnp.where((ranks == r2000) & (kr > 0.0), 1.0, 0.0)
            hi = lax.Precision.HIGHEST
            oy1[...] += lax.dot_general(p, by1, dn, precision=hi,
                                        preferred_element_type=jnp.float32)
            ox1[...] += lax.dot_general(p, bx1, dn, precision=hi,
                                        preferred_element_type=jnp.float32)
            oy2[...] += lax.dot_general(p, by2, dn, precision=hi,
                                        preferred_element_type=jnp.float32)
            ox2[...] += lax.dot_general(p, bx2, dn, precision=hi,
                                        preferred_element_type=jnp.float32)

    out_ref[0] = jnp.concatenate(
        [oy1[...], ox1[...], oy2[...], ox2[...]], axis=1)


def _nms_call(boxes):
    vm = functools.partial(pltpu.VMEM, dtype=jnp.float32)
    return pl.pallas_call(
        _nms_kernel,
        grid=(_B,),
        in_specs=[pl.BlockSpec((1, _PADN, 4), lambda b: (b, 0, 0))],
        out_specs=pl.BlockSpec((1, _PROPOSALS, 4), lambda b: (b, 0, 0)),
        out_shape=jax.ShapeDtypeStruct((_B, _PROPOSALS, 4), jnp.float32),
        scratch_shapes=[vm((_PADN, 1)) for _ in range(6)]
        + [vm((_T, _T))]
        + [vm((_PROPOSALS, 1)) for _ in range(4)],
    )(boxes)


def _decode(boxes, deltas):
    height = boxes[:, 2] - boxes[:, 0]
    width = boxes[:, 3] - boxes[:, 1]
    center_y = boxes[:, 0] + height / 2.0
    center_x = boxes[:, 1] + width / 2.0
    center_y = center_y + deltas[:, 0] * height
    center_x = center_x + deltas[:, 1] * width
    height = height * jnp.exp(deltas[:, 2])
    width = width * jnp.exp(deltas[:, 3])
    y1 = center_y - 0.5 * height
    x1 = center_x - 0.5 * width
    y2 = y1 + height
    x2 = x1 + width
    return jnp.stack([y1, x1, y2, x2], axis=1)


def kernel(rpn_probs, rpn_bbox, anchors):
    std = jnp.asarray(_STD, jnp.float32).reshape(1, 1, 4)
    scores = rpn_probs[:, :, 1]
    dsc = rpn_bbox * std
    table = jnp.concatenate(
        [dsc, anchors, jnp.zeros((_B, _N, 120), jnp.float32)], axis=2)
    table = table.reshape(_B * _N, 128)
    _, ix = lax.top_k(scores, _PRE)
    ixp = jnp.concatenate(
        [ix.astype(jnp.int32),
         jnp.zeros((_B, _PADN - _PRE), jnp.int32)], axis=1)
    ixg = (ixp + (jnp.arange(_B, dtype=jnp.int32) * _N)[:, None]).reshape(-1)
    gathered = _sc_gather(table, ixg).reshape(_B, _PADN, 128)
    boxes = jax.vmap(_decode)(gathered[:, :, 4:8], gathered[:, :, 0:4])
    return _nms_call(boxes)


# speedup vs baseline: 75.6210x; 1.8650x over previous
import functools

import jax
import jax.numpy as jnp
from jax import lax
from jax.experimental import pallas as pl
from jax.experimental.pallas import tpu as pltpu
from jax.experimental.pallas import tpu_sc as plsc

_B = 2
_N = 20000
_PROPOSALS = 2000
_TH = 0.7
_PRE = 6000
_PADN = 6144
_T = 256
_NB = _PADN // _T
_STD = (0.1, 0.1, 0.2, 0.2)


def _sc_gather(table, idx):
    info = plsc.get_sparse_core_info()
    nw = info.num_cores * info.num_subcores
    btot = idx.shape[0]
    bpw = btot // nw
    lanes = table.shape[1]
    mesh = plsc.VectorSubcoreMesh(core_axis_name="c", subcore_axis_name="s")

    @functools.partial(
        pl.kernel,
        mesh=mesh,
        out_type=jax.ShapeDtypeStruct((btot, lanes), jnp.float32),
        scratch_types=[
            pltpu.VMEM((bpw,), jnp.int32),
            pltpu.VMEM((bpw, lanes), jnp.float32),
            pltpu.SemaphoreType.DMA,
        ],
    )
    def k(table_hbm, idx_hbm, out_hbm, idx_v, rows_v, sem):
        wid = lax.axis_index("s") * info.num_cores + lax.axis_index("c")
        base = wid * bpw
        pltpu.sync_copy(idx_hbm.at[pl.ds(base, bpw)], idx_v)
        pltpu.async_copy(table_hbm.at[idx_v], rows_v, sem).wait()
        pltpu.sync_copy(rows_v, out_hbm.at[pl.ds(base, bpw)])

    return k(table, idx)


def _nms_kernel(g_ref, out_ref, y1s, x1s, y2s, x2s, ars, kps, iou_s,
                oy1, ox1, oy2, ox2):
    g = g_ref[0]
    y1 = jnp.clip(g[:, 0:1], 0.0, 1.0)
    x1 = jnp.clip(g[:, 1:2], 0.0, 1.0)
    y2 = jnp.clip(g[:, 2:3], 0.0, 1.0)
    x2 = jnp.clip(g[:, 3:4], 0.0, 1.0)
    area = (y2 - y1) * (x2 - x1)
    y1s[...] = y1
    x1s[...] = x1
    y2s[...] = y2
    x2s[...] = x2
    ars[...] = area

    sub = lax.broadcasted_iota(jnp.int32, (_PADN, 1), 0)
    kps[...] = jnp.where(sub < _PRE, 1.0, 0.0)
    zcol = jnp.zeros((_PROPOSALS, 1), jnp.float32)
    oy1[...] = zcol
    ox1[...] = zcol
    oy2[...] = zcol
    ox2[...] = zcol

    lane = lax.broadcasted_iota(jnp.int32, (1, _T), 1)
    r2000 = lax.broadcasted_iota(
        jnp.int32, (_PROPOSALS, 1), 0).astype(jnp.float32)
    csm = jnp.where(
        lax.broadcasted_iota(jnp.int32, (_T, _T), 0)
        <= lax.broadcasted_iota(jnp.int32, (_T, _T), 1), 1.0, 0.0)
    dn = (((1,), (0,)), ((), ()))

    for b in range(_NB):
        r0 = b * _T
        cnt = jnp.sum(jnp.where(sub < r0, kps[...], 0.0))

        @pl.when(cnt < float(_PROPOSALS))
        def _block(b=b, r0=r0, cnt=cnt):
            by1 = y1s[r0:r0 + _T, :]
            bx1 = x1s[r0:r0 + _T, :]
            by2 = y2s[r0:r0 + _T, :]
            bx2 = x2s[r0:r0 + _T, :]
            bar = ars[r0:r0 + _T, :]
            ry1 = jnp.transpose(by1)
            rx1 = jnp.transpose(bx1)
            ry2 = jnp.transpose(by2)
            rx2 = jnp.transpose(bx2)
            rar = jnp.transpose(bar)

            def chunk_body(c, sup):
                c0 = c * _T
                cy1 = y1s[pl.ds(c0, _T), :]
                cx1 = x1s[pl.ds(c0, _T), :]
                cy2 = y2s[pl.ds(c0, _T), :]
                cx2 = x2s[pl.ds(c0, _T), :]
                car = ars[pl.ds(c0, _T), :]
                ckp = kps[pl.ds(c0, _T), :]
                yy1 = jnp.maximum(cy1, ry1)
                xx1 = jnp.maximum(cx1, rx1)
                yy2 = jnp.minimum(cy2, ry2)
                xx2 = jnp.minimum(cx2, rx2)
                inter = (jnp.maximum(yy2 - yy1, 0.0)
                         * jnp.maximum(xx2 - xx1, 0.0))
                union = car + rar - inter
                iou = inter / (union + 1e-9)
                hit = jnp.where((iou > _TH) & (ckp > 0.0), 1.0, 0.0)
                return jnp.maximum(sup, jnp.max(hit, axis=0, keepdims=True))

            sup = jnp.zeros((1, _T), jnp.float32)
            if b > 0:
                sup = lax.fori_loop(0, b, chunk_body, sup)

            valid = jnp.transpose(kps[r0:r0 + _T, :])
            keep0 = valid * (1.0 - sup)

            yy1 = jnp.maximum(by1, ry1)
            xx1 = jnp.maximum(bx1, rx1)
            yy2 = jnp.minimum(by2, ry2)
            xx2 = jnp.minimum(bx2, rx2)
            inter = jnp.maximum(yy2 - yy1, 0.0) * jnp.maximum(xx2 - xx1, 0.0)
            union = bar + rar - inter
            iou = inter / (union + 1e-9)
            supmask = lax.broadcasted_iota(jnp.int32, (_T, _T), 0) \
                < lax.broadcasted_iota(jnp.int32, (_T, _T), 1)
            iou_s[...] = jnp.where((iou > _TH) & supmask, 1.0, 0.0)

            def fp_cond(c):
                return c[1]

            def fp_body(c):
                k, _ = c
                sup = lax.dot_general(k, iou_s[...], dn,
                                      preferred_element_type=jnp.float32)
                knew = keep0 * jnp.where(sup > 0.0, 0.0, 1.0)
                return knew, jnp.any(knew != k)

            kr, _ = lax.while_loop(fp_cond, fp_body, (keep0, True))
            kps[r0:r0 + _T, :] = jnp.transpose(kr)

            csum = lax.dot_general(kr, csm, dn,
                                   preferred_element_type=jnp.float32)
            ranks = (cnt + csum) - kr
            p = jnp.where((ranks == r2000) & (kr > 0.0), 1.0, 0.0)
            hi = lax.Precision.HIGHEST
            oy1[...] += lax.dot_general(p, by1, dn, precision=hi,
                                        preferred_element_type=jnp.float32)
            ox1[...] += lax.dot_general(p, bx1, dn, precision=hi,
                                        preferred_element_type=jnp.float32)
            oy2[...] += lax.dot_general(p, by2, dn, precision=hi,
                                        preferred_element_type=jnp.float32)
            ox2[...] += lax.dot_general(p, bx2, dn, precision=hi,
                                        preferred_element_type=jnp.float32)

    out_ref[0] = jnp.concatenate(
        [oy1[...], ox1[...], oy2[...], ox2[...]], axis=1)


def _nms_call(boxes):
    vm = functools.partial(pltpu.VMEM, dtype=jnp.float32)
    return pl.pallas_call(
        _nms_kernel,
        grid=(_B,),
        in_specs=[pl.BlockSpec((1, _PADN, 4), lambda b: (b, 0, 0))],
        out_specs=pl.BlockSpec((1, _PROPOSALS, 4), lambda b: (b, 0, 0)),
        out_shape=jax.ShapeDtypeStruct((_B, _PROPOSALS, 4), jnp.float32),
        scratch_shapes=[vm((_PADN, 1)) for _ in range(6)]
        + [vm((_T, _T))]
        + [vm((_PROPOSALS, 1)) for _ in range(4)],
    )(boxes)


def _decode(boxes, deltas):
    height = boxes[:, 2] - boxes[:, 0]
    width = boxes[:, 3] - boxes[:, 1]
    center_y = boxes[:, 0] + height / 2.0
    center_x = boxes[:, 1] + width / 2.0
    center_y = center_y + deltas[:, 0] * height
    center_x = center_x + deltas[:, 1] * width
    height = height * jnp.exp(deltas[:, 2])
    width = width * jnp.exp(deltas[:, 3])
    y1 = center_y - 0.5 * height
    x1 = center_x - 0.5 * width
    y2 = y1 + height
    x2 = x1 + width
    return jnp.stack([y1, x1, y2, x2], axis=1)


def kernel(rpn_probs, rpn_bbox, anchors):
    std = jnp.asarray(_STD, jnp.float32).reshape(1, 1, 4)
    scores = rpn_probs[:, :, 1]
    dsc = rpn_bbox * std
    table = jnp.concatenate(
        [dsc, anchors, jnp.zeros((_B, _N, 120), jnp.float32)], axis=2)
    table = table.reshape(_B * _N, 128)
    _, ix = lax.top_k(scores, _PRE)
    ixp = jnp.concatenate(
        [ix.astype(jnp.int32),
         jnp.zeros((_B, _PADN - _PRE), jnp.int32)], axis=1)
    ixg = (ixp + (jnp.arange(_B, dtype=jnp.int32) * _N)[:, None]).reshape(-1)
    gathered = _sc_gather(table, ixg).reshape(_B, _PADN, 128)
    boxes = jax.vmap(_decode)(gathered[:, :, 4:8], gathered[:, :, 0:4])
    return _nms_call(boxes)


# speedup vs baseline: 104.6388x; 1.3837x over previous
import functools

import jax
import jax.numpy as jnp
from jax import lax
from jax.experimental import pallas as pl
from jax.experimental.pallas import tpu as pltpu
from jax.experimental.pallas import tpu_sc as plsc

_B = 2
_N = 20000
_PROPOSALS = 2000
_TH = 0.7
_PRE = 6000
_PADN = 6144
_T = 256
_NB = _PADN // _T
_STD = (0.1, 0.1, 0.2, 0.2)


def _sc_gather(table, idx):
    info = plsc.get_sparse_core_info()
    nw = info.num_cores * info.num_subcores
    btot = idx.shape[0]
    bpw = btot // nw
    lanes = table.shape[1]
    mesh = plsc.VectorSubcoreMesh(core_axis_name="c", subcore_axis_name="s")

    @functools.partial(
        pl.kernel,
        mesh=mesh,
        out_type=jax.ShapeDtypeStruct((btot, lanes), jnp.float32),
        scratch_types=[
            pltpu.VMEM((bpw,), jnp.int32),
            pltpu.VMEM((bpw, lanes), jnp.float32),
            pltpu.SemaphoreType.DMA,
        ],
    )
    def k(table_hbm, idx_hbm, out_hbm, idx_v, rows_v, sem):
        wid = lax.axis_index("s") * info.num_cores + lax.axis_index("c")
        base = wid * bpw
        pltpu.sync_copy(idx_hbm.at[pl.ds(base, bpw)], idx_v)
        pltpu.async_copy(table_hbm.at[idx_v], rows_v, sem).wait()
        pltpu.sync_copy(rows_v, out_hbm.at[pl.ds(base, bpw)])

    return k(table, idx)


def _nms_kernel(g_ref, out_ref, y1s, x1s, y2s, x2s, ars, kps, iou_s, obox):
    g = g_ref[0]
    y1 = jnp.clip(g[:, 0:1], 0.0, 1.0)
    x1 = jnp.clip(g[:, 1:2], 0.0, 1.0)
    y2 = jnp.clip(g[:, 2:3], 0.0, 1.0)
    x2 = jnp.clip(g[:, 3:4], 0.0, 1.0)
    area = (y2 - y1) * (x2 - x1)
    y1s[...] = y1
    x1s[...] = x1
    y2s[...] = y2
    x2s[...] = x2
    ars[...] = area

    sub = lax.broadcasted_iota(jnp.int32, (_PADN, 1), 0)
    kps[...] = jnp.where(sub < _PRE, 1.0, 0.0)
    obox[...] = jnp.zeros((_PROPOSALS, 4), jnp.float32)

    lane = lax.broadcasted_iota(jnp.int32, (1, _T), 1)
    r2000 = lax.broadcasted_iota(
        jnp.int32, (_PROPOSALS, 1), 0).astype(jnp.float32)
    csm = jnp.where(
        lax.broadcasted_iota(jnp.int32, (_T, _T), 0)
        <= lax.broadcasted_iota(jnp.int32, (_T, _T), 1), 1.0, 0.0)
    dn = (((1,), (0,)), ((), ()))

    for b in range(_NB):
        r0 = b * _T
        cnt = jnp.sum(jnp.where(sub < r0, kps[...], 0.0))

        @pl.when(cnt < float(_PROPOSALS))
        def _block(b=b, r0=r0, cnt=cnt):
            by1 = y1s[r0:r0 + _T, :]
            bx1 = x1s[r0:r0 + _T, :]
            by2 = y2s[r0:r0 + _T, :]
            bx2 = x2s[r0:r0 + _T, :]
            bar = ars[r0:r0 + _T, :]
            ry1 = jnp.transpose(by1)
            rx1 = jnp.transpose(bx1)
            ry2 = jnp.transpose(by2)
            rx2 = jnp.transpose(bx2)
            rar = jnp.transpose(bar)

            def chunk_body(c, sup):
                c0 = c * _T
                cy1 = y1s[pl.ds(c0, _T), :]
                cx1 = x1s[pl.ds(c0, _T), :]
                cy2 = y2s[pl.ds(c0, _T), :]
                cx2 = x2s[pl.ds(c0, _T), :]
                car = ars[pl.ds(c0, _T), :]
                ckp = kps[pl.ds(c0, _T), :]
                yy1 = jnp.maximum(cy1, ry1)
                xx1 = jnp.maximum(cx1, rx1)
                yy2 = jnp.minimum(cy2, ry2)
                xx2 = jnp.minimum(cx2, rx2)
                inter = (jnp.maximum(yy2 - yy1, 0.0)
                         * jnp.maximum(xx2 - xx1, 0.0))
                union = car + rar - inter
                iou = inter / (union + 1e-9)
                hit = jnp.where((iou > _TH) & (ckp > 0.0), 1.0, 0.0)
                return jnp.maximum(sup, jnp.max(hit, axis=0, keepdims=True))

            sup = jnp.zeros((1, _T), jnp.float32)
            if b > 0:
                sup = lax.fori_loop(0, b, chunk_body, sup)

            valid = jnp.transpose(kps[r0:r0 + _T, :])
            keep0 = valid * (1.0 - sup)

            yy1 = jnp.maximum(by1, ry1)
            xx1 = jnp.maximum(bx1, rx1)
            yy2 = jnp.minimum(by2, ry2)
            xx2 = jnp.minimum(bx2, rx2)
            inter = jnp.maximum(yy2 - yy1, 0.0) * jnp.maximum(xx2 - xx1, 0.0)
            union = bar + rar - inter
            iou = inter / (union + 1e-9)
            supmask = lax.broadcasted_iota(jnp.int32, (_T, _T), 0) \
                < lax.broadcasted_iota(jnp.int32, (_T, _T), 1)
            iou_s[...] = jnp.where((iou > _TH) & supmask, 1.0, 0.0)

            def fp_cond(c):
                return c[1]

            def fp_body(c):
                k, _ = c
                sup = lax.dot_general(k, iou_s[...], dn,
                                      preferred_element_type=jnp.float32)
                knew = keep0 * jnp.where(sup > 0.0, 0.0, 1.0)
                return knew, jnp.any(knew != k)

            kr, _ = lax.while_loop(fp_cond, fp_body, (keep0, True))
            kps[r0:r0 + _T, :] = jnp.transpose(kr)

            csum = lax.dot_general(kr, csm, dn,
                                   preferred_element_type=jnp.float32)
            ranks = (cnt + csum) - kr
            p = jnp.where((ranks == r2000) & (kr > 0.0), 1.0, 0.0)
            bb = jnp.concatenate([by1, bx1, by2, bx2], axis=1)
            obox[...] += lax.dot_general(p, bb, dn,
                                         precision=lax.Precision.HIGHEST,
                                         preferred_element_type=jnp.float32)

    out_ref[0] = obox[...]


def _nms_call(boxes):
    vm = functools.partial(pltpu.VMEM, dtype=jnp.float32)
    return pl.pallas_call(
        _nms_kernel,
        grid=(_B,),
        in_specs=[pl.BlockSpec((1, _PADN, 4), lambda b: (b, 0, 0))],
        out_specs=pl.BlockSpec((1, _PROPOSALS, 4), lambda b: (b, 0, 0)),
        out_shape=jax.ShapeDtypeStruct((_B, _PROPOSALS, 4), jnp.float32),
        scratch_shapes=[vm((_PADN, 1)) for _ in range(6)]
        + [vm((_T, _T)), vm((_PROPOSALS, 4))],
    )(boxes)


def _decode(boxes, deltas):
    height = boxes[:, 2] - boxes[:, 0]
    width = boxes[:, 3] - boxes[:, 1]
    center_y = boxes[:, 0] + height / 2.0
    center_x = boxes[:, 1] + width / 2.0
    center_y = center_y + deltas[:, 0] * height
    center_x = center_x + deltas[:, 1] * width
    height = height * jnp.exp(deltas[:, 2])
    width = width * jnp.exp(deltas[:, 3])
    y1 = center_y - 0.5 * height
    x1 = center_x - 0.5 * width
    y2 = y1 + height
    x2 = x1 + width
    return jnp.stack([y1, x1, y2, x2], axis=1)


def kernel(rpn_probs, rpn_bbox, anchors):
    std = jnp.asarray(_STD, jnp.float32).reshape(1, 1, 4)
    scores = rpn_probs[:, :, 1]
    dsc = rpn_bbox * std
    table = jnp.concatenate(
        [dsc, anchors, jnp.zeros((_B, _N, 120), jnp.float32)], axis=2)
    table = table.reshape(_B * _N, 128)
    _, ix = lax.top_k(scores, _PRE)
    ixp = jnp.concatenate(
        [ix.astype(jnp.int32),
         jnp.zeros((_B, _PADN - _PRE), jnp.int32)], axis=1)
    ixg = (ixp + (jnp.arange(_B, dtype=jnp.int32) * _N)[:, None]).reshape(-1)
    gathered = _sc_gather(table, ixg).reshape(_B, _PADN, 128)
    boxes = jax.vmap(_decode)(gathered[:, :, 4:8], gathered[:, :, 0:4])
    return _nms_call(boxes)


# speedup vs baseline: 108.6604x; 1.0384x over previous
import functools

import jax
import jax.numpy as jnp
from jax import lax
from jax.experimental import pallas as pl
from jax.experimental.pallas import tpu as pltpu
from jax.experimental.pallas import tpu_sc as plsc

_B = 2
_N = 20000
_PROPOSALS = 2000
_TH = 0.7
_PRE = 6000
_PADN = 6144
_T = 256
_NB = _PADN // _T
_STD = (0.1, 0.1, 0.2, 0.2)


def _sc_gather(table, idx):
    info = plsc.get_sparse_core_info()
    nw = info.num_cores * info.num_subcores
    btot = idx.shape[0]
    bpw = btot // nw
    lanes = table.shape[1]
    mesh = plsc.VectorSubcoreMesh(core_axis_name="c", subcore_axis_name="s")

    @functools.partial(
        pl.kernel,
        mesh=mesh,
        out_type=jax.ShapeDtypeStruct((btot, lanes), jnp.float32),
        scratch_types=[
            pltpu.VMEM((bpw,), jnp.int32),
            pltpu.VMEM((bpw, lanes), jnp.float32),
            pltpu.SemaphoreType.DMA,
        ],
    )
    def k(table_hbm, idx_hbm, out_hbm, idx_v, rows_v, sem):
        wid = lax.axis_index("s") * info.num_cores + lax.axis_index("c")
        base = wid * bpw
        pltpu.sync_copy(idx_hbm.at[pl.ds(base, bpw)], idx_v)
        pltpu.async_copy(table_hbm.at[idx_v], rows_v, sem).wait()
        pltpu.sync_copy(rows_v, out_hbm.at[pl.ds(base, bpw)])

    return k(table, idx)


def _nms_kernel(g_ref, out_ref, y1s, x1s, y2s, x2s, ars, kps, iou_s, obox,
                cnt_s):
    g = g_ref[0]
    y1 = jnp.clip(g[:, 0:1], 0.0, 1.0)
    x1 = jnp.clip(g[:, 1:2], 0.0, 1.0)
    y2 = jnp.clip(g[:, 2:3], 0.0, 1.0)
    x2 = jnp.clip(g[:, 3:4], 0.0, 1.0)
    area = (y2 - y1) * (x2 - x1)
    y1s[...] = y1
    x1s[...] = x1
    y2s[...] = y2
    x2s[...] = x2
    ars[...] = area

    sub = lax.broadcasted_iota(jnp.int32, (_PADN, 1), 0)
    kps[...] = jnp.where(sub < _PRE, 1.0, 0.0)
    obox[...] = jnp.zeros((_PROPOSALS, 4), jnp.float32)
    cnt_s[0] = 0.0

    lane = lax.broadcasted_iota(jnp.int32, (1, _T), 1)
    r2000 = lax.broadcasted_iota(
        jnp.int32, (_PROPOSALS, 1), 0).astype(jnp.float32)
    csm = jnp.where(
        lax.broadcasted_iota(jnp.int32, (_T, _T), 0)
        <= lax.broadcasted_iota(jnp.int32, (_T, _T), 1), 1.0, 0.0)
    dn = (((1,), (0,)), ((), ()))

    for b in range(_NB):
        r0 = b * _T
        cnt = cnt_s[0]

        @pl.when(cnt < float(_PROPOSALS))
        def _block(b=b, r0=r0, cnt=cnt):
            by1 = y1s[r0:r0 + _T, :]
            bx1 = x1s[r0:r0 + _T, :]
            by2 = y2s[r0:r0 + _T, :]
            bx2 = x2s[r0:r0 + _T, :]
            bar = ars[r0:r0 + _T, :]
            ry1 = jnp.transpose(by1)
            rx1 = jnp.transpose(bx1)
            ry2 = jnp.transpose(by2)
            rx2 = jnp.transpose(bx2)
            rar = jnp.transpose(bar)

            def chunk_body(c, sup):
                c0 = c * _T
                cy1 = y1s[pl.ds(c0, _T), :]
                cx1 = x1s[pl.ds(c0, _T), :]
                cy2 = y2s[pl.ds(c0, _T), :]
                cx2 = x2s[pl.ds(c0, _T), :]
                car = ars[pl.ds(c0, _T), :]
                ckp = kps[pl.ds(c0, _T), :]
                yy1 = jnp.maximum(cy1, ry1)
                xx1 = jnp.maximum(cx1, rx1)
                yy2 = jnp.minimum(cy2, ry2)
                xx2 = jnp.minimum(cx2, rx2)
                inter = (jnp.maximum(yy2 - yy1, 0.0)
                         * jnp.maximum(xx2 - xx1, 0.0))
                union = car + rar - inter
                iou = inter / (union + 1e-9)
                hit = jnp.where((iou > _TH) & (ckp > 0.0), 1.0, 0.0)
                return jnp.maximum(sup, jnp.max(hit, axis=0, keepdims=True))

            sup = jnp.zeros((1, _T), jnp.float32)
            if b > 0:
                sup = lax.fori_loop(0, b, chunk_body, sup)

            valid = jnp.transpose(kps[r0:r0 + _T, :])
            keep0 = valid * (1.0 - sup)

            yy1 = jnp.maximum(by1, ry1)
            xx1 = jnp.maximum(bx1, rx1)
            yy2 = jnp.minimum(by2, ry2)
            xx2 = jnp.minimum(bx2, rx2)
            inter = jnp.maximum(yy2 - yy1, 0.0) * jnp.maximum(xx2 - xx1, 0.0)
            union = bar + rar - inter
            iou = inter / (union + 1e-9)
            supmask = lax.broadcasted_iota(jnp.int32, (_T, _T), 0) \
                < lax.broadcasted_iota(jnp.int32, (_T, _T), 1)
            iou_s[...] = jnp.where((iou > _TH) & supmask, 1.0, 0.0)

            def fp_cond(c):
                return c[1]

            def fp_body(c):
                k, _ = c
                sup = lax.dot_general(k, iou_s[...], dn,
                                      preferred_element_type=jnp.float32)
                knew = keep0 * jnp.where(sup > 0.0, 0.0, 1.0)
                return knew, jnp.any(knew != k)

            kr, _ = lax.while_loop(fp_cond, fp_body, (keep0, True))
            kps[r0:r0 + _T, :] = jnp.transpose(kr)
            cnt_s[0] = cnt + jnp.sum(kr)

            csum = lax.dot_general(kr, csm, dn,
                                   preferred_element_type=jnp.float32)
            ranks = (cnt + csum) - kr
            p = jnp.where((ranks == r2000) & (kr > 0.0), 1.0, 0.0)
            bb = jnp.concatenate([by1, bx1, by2, bx2], axis=1)
            obox[...] += lax.dot_general(p, bb, dn,
                                         precision=lax.Precision.HIGHEST,
                                         preferred_element_type=jnp.float32)

    out_ref[0] = obox[...]


def _nms_call(boxes):
    vm = functools.partial(pltpu.VMEM, dtype=jnp.float32)
    return pl.pallas_call(
        _nms_kernel,
        grid=(_B,),
        in_specs=[pl.BlockSpec((1, _PADN, 4), lambda b: (b, 0, 0))],
        out_specs=pl.BlockSpec((1, _PROPOSALS, 4), lambda b: (b, 0, 0)),
        out_shape=jax.ShapeDtypeStruct((_B, _PROPOSALS, 4), jnp.float32),
        scratch_shapes=[vm((_PADN, 1)) for _ in range(6)]
        + [vm((_T, _T)), vm((_PROPOSALS, 4)),
           pltpu.SMEM((1,), jnp.float32)],
        compiler_params=pltpu.CompilerParams(
            dimension_semantics=("parallel",)),
    )(boxes)


def _decode(boxes, deltas):
    height = boxes[:, 2] - boxes[:, 0]
    width = boxes[:, 3] - boxes[:, 1]
    center_y = boxes[:, 0] + height / 2.0
    center_x = boxes[:, 1] + width / 2.0
    center_y = center_y + deltas[:, 0] * height
    center_x = center_x + deltas[:, 1] * width
    height = height * jnp.exp(deltas[:, 2])
    width = width * jnp.exp(deltas[:, 3])
    y1 = center_y - 0.5 * height
    x1 = center_x - 0.5 * width
    y2 = y1 + height
    x2 = x1 + width
    return jnp.stack([y1, x1, y2, x2], axis=1)


def kernel(rpn_probs, rpn_bbox, anchors):
    std = jnp.asarray(_STD, jnp.float32).reshape(1, 1, 4)
    scores = rpn_probs[:, :, 1]
    dsc = rpn_bbox * std
    table = jnp.concatenate(
        [dsc, anchors, jnp.zeros((_B, _N, 120), jnp.float32)], axis=2)
    table = table.reshape(_B * _N, 128)
    _, ix = lax.top_k(scores, _PRE)
    ixp = jnp.concatenate(
        [ix.astype(jnp.int32),
         jnp.zeros((_B, _PADN - _PRE), jnp.int32)], axis=1)
    ixg = (ixp + (jnp.arange(_B, dtype=jnp.int32) * _N)[:, None]).reshape(-1)
    gathered = _sc_gather(table, ixg).reshape(_B, _PADN, 128)
    boxes = jax.vmap(_decode)(gathered[:, :, 4:8], gathered[:, :, 0:4])
    return _nms_call(boxes)
